# baseline (speedup 1.0000x reference)
import jax
import jax.numpy as jnp
from jax import lax
from jax.experimental import pallas as pl
from jax.experimental.pallas import tpu as pltpu
from jax.experimental.pallas import tpu_sc as plsc

N_NODES = 10000
IN_CH = 256
HID_CH = 64
HID_PAD = 128

NC = 2
NS = 16
NW = NC * NS
N_PAD = 10240
RPT = N_PAD // NS
E_PAD = 163840
EPW = E_PAD // NW
IB = 128

MC = 128
KR = MC // IB
NMC = EPW // MC
SUP = 8
NMC_A = 48
NMC_B = 80 - NMC_A
EDGES_A = NS * NMC_A * MC
MC1 = 1024
KR1 = MC1 // IB
NMC1 = EPW // MC1
NMC1_A = 6
NMC1_B = 10 - NMC1_A
EDGES1_A = NS * NMC1_A * MC1


def _sc_mesh():
    return plsc.VectorSubcoreMesh(core_axis_name="c", subcore_axis_name="s")


def _zero_acc_and_barrier(z_hbm, acc_sh, s):
    pltpu.sync_copy(z_hbm, acc_sh.at[pl.ds(s * RPT, RPT)])
    plsc.subcore_barrier()


def _writeback(acc_sh, out_hbm, c, s):
    plsc.subcore_barrier()
    pltpu.sync_copy(acc_sh.at[pl.ds(s * RPT, RPT)],
                    out_hbm.at[c, pl.ds(s * RPT, RPT)])


def _deg_body(col2_hbm, ew_hbm, z1_hbm, out_hbm, cidx2, ew2, acc_sh, ssem):
    c = lax.axis_index("c")
    s = lax.axis_index("s")
    wid = c * NS + s
    _zero_acc_and_barrier(z1_hbm, acc_sh, s)
    base = wid * EPW
    base_rows = wid * (EPW // IB)

    def step(k, carry):
        b = k % 2
        bn = 1 - b

        @pl.when(k > 0)
        def _():
            for j in range(KR1):
                pltpu.make_async_copy(
                    ew2.at[pl.ds((bn * KR1 + j) * IB, IB)],
                    acc_sh.at[cidx2.at[bn * KR1 + j]], ssem).wait()

        rs = base_rows + k * KR1
        pltpu.sync_copy(col2_hbm.at[pl.ds(rs, KR1)],
                        cidx2.at[pl.ds(b * KR1, KR1)])
        pltpu.sync_copy(ew_hbm.at[pl.ds(base + k * MC1, MC1)],
                        ew2.at[pl.ds(b * MC1, MC1)])
        for j in range(KR1):
            pltpu.async_copy(ew2.at[pl.ds((b * KR1 + j) * IB, IB)],
                             acc_sh.at[cidx2.at[b * KR1 + j]], ssem, add=True)
        return carry

    lax.fori_loop(0, NMC1, step, 0)
    bl = (NMC1 - 1) % 2
    for j in range(KR1):
        pltpu.make_async_copy(
            ew2.at[pl.ds((bl * KR1 + j) * IB, IB)],
            acc_sh.at[cidx2.at[bl * KR1 + j]], ssem).wait()
    _writeback(acc_sh, out_hbm, c, s)


def _deg_call(col2, ew_p, z1):
    k = pl.kernel(
        _deg_body,
        mesh=_sc_mesh(),
        out_type=jax.ShapeDtypeStruct((NC, N_PAD), jnp.float32),
        scratch_types=[
            pltpu.VMEM((2 * KR1, IB), jnp.int32),
            pltpu.VMEM((2 * MC1,), jnp.float32),
            pltpu.VMEM_SHARED((N_PAD,), jnp.float32),
            pltpu.SemaphoreType.DMA,
        ],
    )
    return k(col2, ew_p, z1)


def _mp64_body(hp_hbm, row2_hbm, col2_hbm, ew_hbm, z2_hbm, out_hbm,
               ridx2, cidx2, ew2, gdst, msg, acc_sh, gsem, ssem, isem):
    c = lax.axis_index("c")
    s = lax.axis_index("s")
    _zero_acc_and_barrier(z2_hbm, acc_sh, s)
    nmc = jnp.where(c == 0, NMC_A, NMC_B)
    base = jnp.where(c == 0, s * (NMC_A * MC),
                     EDGES_A + s * (NMC_B * MC))
    base_rows = base // IB

    def idx_copies(k, sl3):
        rs = base_rows + k
        return (
            pltpu.make_async_copy(row2_hbm.at[pl.ds(rs, 1)],
                                  ridx2.at[pl.ds(sl3, 1)], isem),
            pltpu.make_async_copy(col2_hbm.at[pl.ds(rs, 1)],
                                  cidx2.at[pl.ds(sl3, 1)], isem),
            pltpu.make_async_copy(ew_hbm.at[pl.ds(base + k * MC, MC)],
                                  ew2.at[pl.ds(sl3 * MC, MC)], isem),
        )

    def issue_idx(k, sl3):
        for cp in idx_copies(k, sl3):
            cp.start()

    def drain_idx(k, sl3):
        for cp in idx_copies(k, sl3):
            cp.wait()

    def issue_gather(sl3, b):
        pltpu.async_copy(hp_hbm.at[ridx2.at[sl3]],
                         msg.at[pl.ds(b * MC, MC)], gsem)

    issue_idx(0, 0)
    issue_idx(1, 1)
    drain_idx(0, 0)
    issue_gather(0, 0)

    def step(k, carry):
        b = k % 2
        bn = 1 - b
        sl3 = k % 3
        sln = (k + 1) % 3

        @pl.when(k > 0)
        def _():
            pltpu.make_async_copy(
                msg.at[pl.ds(bn * MC, MC)],
                acc_sh.at[cidx2.at[(k - 1) % 3]], ssem).wait()

        @pl.when(k + 2 < nmc)
        def _():
            issue_idx(k + 2, (k + 2) % 3)

        @pl.when(k + 1 < nmc)
        def _():
            drain_idx(k + 1, sln)
            issue_gather(sln, bn)

        pltpu.make_async_copy(hp_hbm.at[ridx2.at[sl3]],
                              msg.at[pl.ds(b * MC, MC)], gsem).wait()

        def sgrp(t, cc):
            ew16 = ew2[pl.ds(sl3 * MC + t * 16, 16)]
            for jj in range(16):
                w = ew16[jj]
                i = b * MC + t * 16 + jj
                for g in range(HID_CH // 16):
                    sl = pl.ds(g * 16, 16)
                    msg[i, sl] = msg[i, sl] * w
            return cc

        lax.fori_loop(0, MC // 16, sgrp, 0)

        pltpu.async_copy(msg.at[pl.ds(b * MC, MC)],
                         acc_sh.at[cidx2.at[sl3]], ssem, add=True)
        return carry

    lax.fori_loop(0, nmc, step, 0)
    bl = (nmc - 1) % 2
    pltpu.make_async_copy(
        msg.at[pl.ds(bl * MC, MC)],
        acc_sh.at[cidx2.at[(nmc - 1) % 3]], ssem).wait()
    _writeback(acc_sh, out_hbm, c, s)


def _mp64_call(hp, row2, col2, ew_p, z2):
    k = pl.kernel(
        _mp64_body,
        mesh=_sc_mesh(),
        out_type=jax.ShapeDtypeStruct((NC, N_PAD, HID_PAD), jnp.float32),
        scratch_types=[
            pltpu.VMEM((3, IB), jnp.int32),
            pltpu.VMEM((3, IB), jnp.int32),
            pltpu.VMEM((3 * MC,), jnp.float32),
            pltpu.VMEM((1, IB), jnp.int32),
            pltpu.VMEM((2 * MC, HID_PAD), jnp.float32),
            pltpu.VMEM_SHARED((N_PAD, HID_PAD), jnp.float32),
            pltpu.SemaphoreType.DMA,
            pltpu.SemaphoreType.DMA,
            pltpu.SemaphoreType.DMA,
        ],
    )
    return k(hp, row2, col2, ew_p, z2)


def _mp1_body(zp_hbm, row2_hbm, col2_hbm, ew_hbm, z1_hbm, out_hbm,
              ridx2, cidx2, ew2, val, acc_sh, gsem, ssem, isem):
    c = lax.axis_index("c")
    s = lax.axis_index("s")
    _zero_acc_and_barrier(z1_hbm, acc_sh, s)
    nmc = jnp.where(c == 0, NMC1_A, NMC1_B)
    base = jnp.where(c == 0, s * (NMC1_A * MC1),
                     EDGES1_A + s * (NMC1_B * MC1))
    base_rows = base // IB

    def idx_copies(k, sl3):
        rs = pl.multiple_of(base_rows + k * KR1, 8)
        return (
            pltpu.make_async_copy(row2_hbm.at[pl.ds(rs, KR1)],
                                  ridx2.at[pl.ds(sl3 * KR1, KR1)], isem),
            pltpu.make_async_copy(col2_hbm.at[pl.ds(rs, KR1)],
                                  cidx2.at[pl.ds(sl3 * KR1, KR1)], isem),
            pltpu.make_async_copy(ew_hbm.at[pl.ds(base + k * MC1, MC1)],
                                  ew2.at[pl.ds(sl3 * MC1, MC1)], isem),
        )

    def issue_idx(k, sl3):
        for cp in idx_copies(k, sl3):
            cp.start()

    def drain_idx(k, sl3):
        for cp in idx_copies(k, sl3):
            cp.wait()

    def issue_gather(sl3, b):
        for j in range(KR1):
            pltpu.async_copy(zp_hbm.at[ridx2.at[sl3 * KR1 + j]],
                             val.at[pl.ds((b * KR1 + j) * IB, IB)], gsem)

    issue_idx(0, 0)
    issue_idx(1, 1)
    drain_idx(0, 0)
    issue_gather(0, 0)

    def step(k, carry):
        b = k % 2
        bn = 1 - b
        sl3 = k % 3
        sln = (k + 1) % 3

        @pl.when(k > 0)
        def _():
            for j in range(KR1):
                pltpu.make_async_copy(
                    val.at[pl.ds((bn * KR1 + j) * IB, IB)],
                    acc_sh.at[cidx2.at[((k - 1) % 3) * KR1 + j]], ssem).wait()

        @pl.when(k + 2 < nmc)
        def _():
            issue_idx(k + 2, (k + 2) % 3)

        @pl.when(k + 1 < nmc)
        def _():
            drain_idx(k + 1, sln)
            issue_gather(sln, bn)

        for j in range(KR1):
            pltpu.make_async_copy(zp_hbm.at[ridx2.at[sl3 * KR1 + j]],
                                  val.at[pl.ds((b * KR1 + j) * IB, IB)],
                                  gsem).wait()

        for j in range(MC1 // 16):
            slv = pl.ds(b * MC1 + j * 16, 16)
            sle = pl.ds(sl3 * MC1 + j * 16, 16)
            val[slv] = val[slv] * ew2[sle]

        for j in range(KR1):
            pltpu.async_copy(val.at[pl.ds((b * KR1 + j) * IB, IB)],
                             acc_sh.at[cidx2.at[sl3 * KR1 + j]], ssem, add=True)
        return carry

    lax.fori_loop(0, nmc, step, 0)
    bl = (nmc - 1) % 2
    sl_last = (nmc - 1) % 3
    for j in range(KR1):
        pltpu.make_async_copy(
            val.at[pl.ds((bl * KR1 + j) * IB, IB)],
            acc_sh.at[cidx2.at[sl_last * KR1 + j]], ssem).wait()
    _writeback(acc_sh, out_hbm, c, s)


def _mp1_call(zp, row2, col2, ew_p, z1):
    k = pl.kernel(
        _mp1_body,
        mesh=_sc_mesh(),
        out_type=jax.ShapeDtypeStruct((NC, N_PAD), jnp.float32),
        scratch_types=[
            pltpu.VMEM((3 * KR1, IB), jnp.int32),
            pltpu.VMEM((3 * KR1, IB), jnp.int32),
            pltpu.VMEM((3 * MC1,), jnp.float32),
            pltpu.VMEM((2 * MC1,), jnp.float32),
            pltpu.VMEM_SHARED((N_PAD,), jnp.float32),
            pltpu.SemaphoreType.DMA,
            pltpu.SemaphoreType.DMA,
            pltpu.SemaphoreType.DMA,
        ],
    )
    return k(zp, row2, col2, ew_p, z1)


_BN = 1000


def _tc_b_body(x_ref, w1_ref, degp_ref, hp_ref, dis_ref):
    h = jnp.dot(x_ref[...], w1_ref[...], preferred_element_type=jnp.float32)
    deg = degp_ref[:, 0] + degp_ref[:, 1] + 1.0
    dis = lax.rsqrt(deg)
    hp_ref[...] = h * dis[:, None]
    dis_ref[...] = dis[:, None]


def _b_call(x, W1p, degp_t):
    return pl.pallas_call(
        _tc_b_body,
        grid=(N_NODES // _BN,),
        in_specs=[
            pl.BlockSpec((_BN, IN_CH), lambda i: (i, 0)),
            pl.BlockSpec((IN_CH, HID_PAD), lambda i: (0, 0)),
            pl.BlockSpec((_BN, NC), lambda i: (i, 0)),
        ],
        out_specs=[
            pl.BlockSpec((_BN, HID_PAD), lambda i: (i, 0)),
            pl.BlockSpec((_BN, 1), lambda i: (i, 0)),
        ],
        out_shape=[
            jax.ShapeDtypeStruct((N_NODES, HID_PAD), jnp.float32),
            jax.ShapeDtypeStruct((N_NODES, 1), jnp.float32),
        ],
    )(x, W1p, degp_t)


def _tc_d_body(accp_ref, hp_ref, dis_ref, b1_ref, w2_ref, zp_ref):
    acc = (accp_ref[0, :, :HID_CH] + accp_ref[1, :, :HID_CH]
           + hp_ref[:, :HID_CH])
    out1 = acc * dis_ref[...] + b1_ref[...]
    h1 = jnp.maximum(out1, 0.0)
    z = jnp.dot(h1, w2_ref[...], preferred_element_type=jnp.float32)
    zp_ref[...] = z * dis_ref[...]


def _d_call(accp, hp, dis, b1r, W2):
    return pl.pallas_call(
        _tc_d_body,
        grid=(N_NODES // _BN,),
        in_specs=[
            pl.BlockSpec((NC, _BN, HID_PAD), lambda i: (0, i, 0)),
            pl.BlockSpec((_BN, HID_PAD), lambda i: (i, 0)),
            pl.BlockSpec((_BN, 1), lambda i: (i, 0)),
            pl.BlockSpec((1, HID_CH), lambda i: (0, 0)),
            pl.BlockSpec((HID_CH, 1), lambda i: (0, 0)),
        ],
        out_specs=pl.BlockSpec((_BN, 1), lambda i: (i, 0)),
        out_shape=jax.ShapeDtypeStruct((N_NODES, 1), jnp.float32),
    )(accp, hp, dis, b1r, W2)


def _tc_f_body(acc2p_ref, zp_ref, dis_ref, b2_ref, out_ref):
    a = acc2p_ref[:, 0] + acc2p_ref[:, 1]
    out_ref[...] = dis_ref[...] * (a[:, None] + zp_ref[...]) + b2_ref[...]


def _f_call(acc2p_t, zp, dis, b2r):
    return pl.pallas_call(
        _tc_f_body,
        grid=(N_NODES // _BN,),
        in_specs=[
            pl.BlockSpec((_BN, NC), lambda i: (i, 0)),
            pl.BlockSpec((_BN, 1), lambda i: (i, 0)),
            pl.BlockSpec((_BN, 1), lambda i: (i, 0)),
            pl.BlockSpec((1, 1), lambda i: (0, 0)),
        ],
        out_specs=pl.BlockSpec((_BN, 1), lambda i: (i, 0)),
        out_shape=jax.ShapeDtypeStruct((N_NODES, 1), jnp.float32),
    )(acc2p_t, zp, dis, b2r)


def kernel(x, edge_index, edge_weight, W1, b1, W2, b2):
    row = edge_index[0].astype(jnp.int32)
    col = edge_index[1].astype(jnp.int32)
    ew = edge_weight.astype(jnp.float32)
    pad = E_PAD - row.shape[0]
    row2 = jnp.concatenate([row, jnp.zeros((pad,), jnp.int32)]).reshape(-1, IB)
    col2 = jnp.concatenate([col, jnp.zeros((pad,), jnp.int32)]).reshape(-1, IB)
    ew_p = jnp.concatenate([ew, jnp.zeros((pad,), jnp.float32)])
    z1 = jnp.zeros((RPT,), jnp.float32)
    z2 = jnp.zeros((RPT, HID_PAD), jnp.float32)

    W1p = jnp.concatenate(
        [W1, jnp.zeros((IN_CH, HID_PAD - HID_CH), jnp.float32)], axis=1)
    degp = _deg_call(col2, ew_p, z1)
    hp, dis = _b_call(x, W1p, degp.T)
    accp = _mp64_call(hp, row2, col2, ew_p, z2)
    zp = _d_call(accp, hp, dis, b1.reshape(1, HID_CH), W2)
    acc2p = _mp1_call(zp.reshape(-1), row2, col2, ew_p, z1)
    out = _f_call(acc2p.T, zp, dis, b2.reshape(1, 1))
    return out.reshape(-1)


# speedup vs baseline: 1.0345x; 1.0345x over previous
import jax
import jax.numpy as jnp
from jax import lax
from jax.experimental import pallas as pl
from jax.experimental.pallas import tpu as pltpu
from jax.experimental.pallas import tpu_sc as plsc

N_NODES = 10000
IN_CH = 256
HID_CH = 64
HID_PAD = 128

NC = 2
NS = 16
NW = NC * NS
N_PAD = 10240
RPT = N_PAD // NS
E_PAD = 163840
EPW = E_PAD // NW
IB = 128

MC = 128
KR = MC // IB
NMC = EPW // MC
SUP = 8
NMC_A = 56
NMC_B = 80 - NMC_A
EDGES_A = NS * NMC_A * MC
MC1 = 1024
KR1 = MC1 // IB
NMC1 = EPW // MC1
NMC1_A = 6
NMC1_B = 10 - NMC1_A
EDGES1_A = NS * NMC1_A * MC1


def _sc_mesh():
    return plsc.VectorSubcoreMesh(core_axis_name="c", subcore_axis_name="s")


def _zero_acc_and_barrier(z_hbm, acc_sh, s):
    pltpu.sync_copy(z_hbm, acc_sh.at[pl.ds(s * RPT, RPT)])
    plsc.subcore_barrier()


def _writeback(acc_sh, out_hbm, c, s):
    plsc.subcore_barrier()
    pltpu.sync_copy(acc_sh.at[pl.ds(s * RPT, RPT)],
                    out_hbm.at[c, pl.ds(s * RPT, RPT)])


def _deg_body(col2_hbm, ew_hbm, z1_hbm, out_hbm, cidx2, ew2, acc_sh, ssem):
    c = lax.axis_index("c")
    s = lax.axis_index("s")
    wid = c * NS + s
    _zero_acc_and_barrier(z1_hbm, acc_sh, s)
    base = wid * EPW
    base_rows = wid * (EPW // IB)

    def step(k, carry):
        b = k % 2
        bn = 1 - b

        @pl.when(k > 0)
        def _():
            for j in range(KR1):
                pltpu.make_async_copy(
                    ew2.at[pl.ds((bn * KR1 + j) * IB, IB)],
                    acc_sh.at[cidx2.at[bn * KR1 + j]], ssem).wait()

        rs = base_rows + k * KR1
        pltpu.sync_copy(col2_hbm.at[pl.ds(rs, KR1)],
                        cidx2.at[pl.ds(b * KR1, KR1)])
        pltpu.sync_copy(ew_hbm.at[pl.ds(base + k * MC1, MC1)],
                        ew2.at[pl.ds(b * MC1, MC1)])
        for j in range(KR1):
            pltpu.async_copy(ew2.at[pl.ds((b * KR1 + j) * IB, IB)],
                             acc_sh.at[cidx2.at[b * KR1 + j]], ssem, add=True)
        return carry

    lax.fori_loop(0, NMC1, step, 0)
    bl = (NMC1 - 1) % 2
    for j in range(KR1):
        pltpu.make_async_copy(
            ew2.at[pl.ds((bl * KR1 + j) * IB, IB)],
            acc_sh.at[cidx2.at[bl * KR1 + j]], ssem).wait()
    _writeback(acc_sh, out_hbm, c, s)


def _deg_call(col2, ew_p, z1):
    k = pl.kernel(
        _deg_body,
        mesh=_sc_mesh(),
        out_type=jax.ShapeDtypeStruct((NC, N_PAD), jnp.float32),
        scratch_types=[
            pltpu.VMEM((2 * KR1, IB), jnp.int32),
            pltpu.VMEM((2 * MC1,), jnp.float32),
            pltpu.VMEM_SHARED((N_PAD,), jnp.float32),
            pltpu.SemaphoreType.DMA,
        ],
    )
    return k(col2, ew_p, z1)


def _mp64_body(hp_hbm, row2_hbm, col2_hbm, ew_hbm, z2_hbm, out_hbm,
               ridx2, cidx2, ew2, gdst, msg, acc_sh, gsem, ssem, isem):
    c = lax.axis_index("c")
    s = lax.axis_index("s")
    _zero_acc_and_barrier(z2_hbm, acc_sh, s)
    nmc = jnp.where(c == 0, NMC_A, NMC_B)
    base = jnp.where(c == 0, s * (NMC_A * MC),
                     EDGES_A + s * (NMC_B * MC))
    base_rows = base // IB

    def idx_copies(k, sl3):
        rs = base_rows + k
        return (
            pltpu.make_async_copy(row2_hbm.at[pl.ds(rs, 1)],
                                  ridx2.at[pl.ds(sl3, 1)], isem),
            pltpu.make_async_copy(col2_hbm.at[pl.ds(rs, 1)],
                                  cidx2.at[pl.ds(sl3, 1)], isem),
            pltpu.make_async_copy(ew_hbm.at[pl.ds(base + k * MC, MC)],
                                  ew2.at[pl.ds(sl3 * MC, MC)], isem),
        )

    def issue_idx(k, sl3):
        for cp in idx_copies(k, sl3):
            cp.start()

    def drain_idx(k, sl3):
        for cp in idx_copies(k, sl3):
            cp.wait()

    def issue_gather(sl3, b):
        pltpu.async_copy(hp_hbm.at[ridx2.at[sl3]],
                         msg.at[pl.ds(b * MC, MC)], gsem)

    issue_idx(0, 0)
    issue_idx(1, 1)
    drain_idx(0, 0)
    issue_gather(0, 0)

    def step(k, carry):
        b = k % 2
        bn = 1 - b
        sl3 = k % 3
        sln = (k + 1) % 3

        @pl.when(k > 0)
        def _():
            pltpu.make_async_copy(
                msg.at[pl.ds(bn * MC, MC)],
                acc_sh.at[cidx2.at[(k - 1) % 3]], ssem).wait()

        @pl.when(k + 2 < nmc)
        def _():
            issue_idx(k + 2, (k + 2) % 3)

        @pl.when(k + 1 < nmc)
        def _():
            drain_idx(k + 1, sln)
            issue_gather(sln, bn)

        pltpu.make_async_copy(hp_hbm.at[ridx2.at[sl3]],
                              msg.at[pl.ds(b * MC, MC)], gsem).wait()

        def sgrp(t, cc):
            ew16 = ew2[pl.ds(sl3 * MC + t * 16, 16)]
            for jj in range(16):
                w = ew16[jj]
                i = b * MC + t * 16 + jj
                for g in range(HID_CH // 16):
                    sl = pl.ds(g * 16, 16)
                    msg[i, sl] = msg[i, sl] * w
            return cc

        lax.fori_loop(0, MC // 16, sgrp, 0)

        pltpu.async_copy(msg.at[pl.ds(b * MC, MC)],
                         acc_sh.at[cidx2.at[sl3]], ssem, add=True)
        return carry

    lax.fori_loop(0, nmc, step, 0)
    bl = (nmc - 1) % 2
    pltpu.make_async_copy(
        msg.at[pl.ds(bl * MC, MC)],
        acc_sh.at[cidx2.at[(nmc - 1) % 3]], ssem).wait()
    _writeback(acc_sh, out_hbm, c, s)


def _mp64_call(hp, row2, col2, ew_p, z2):
    k = pl.kernel(
        _mp64_body,
        mesh=_sc_mesh(),
        out_type=jax.ShapeDtypeStruct((NC, N_PAD, HID_PAD), jnp.float32),
        scratch_types=[
            pltpu.VMEM((3, IB), jnp.int32),
            pltpu.VMEM((3, IB), jnp.int32),
            pltpu.VMEM((3 * MC,), jnp.float32),
            pltpu.VMEM((1, IB), jnp.int32),
            pltpu.VMEM((2 * MC, HID_PAD), jnp.float32),
            pltpu.VMEM_SHARED((N_PAD, HID_PAD), jnp.float32),
            pltpu.SemaphoreType.DMA,
            pltpu.SemaphoreType.DMA,
            pltpu.SemaphoreType.DMA,
        ],
    )
    return k(hp, row2, col2, ew_p, z2)


def _mp1_body(zp_hbm, row2_hbm, col2_hbm, ew_hbm, z1_hbm, out_hbm,
              ridx2, cidx2, ew2, val, acc_sh, gsem, ssem, isem):
    c = lax.axis_index("c")
    s = lax.axis_index("s")
    _zero_acc_and_barrier(z1_hbm, acc_sh, s)
    nmc = jnp.where(c == 0, NMC1_A, NMC1_B)
    base = jnp.where(c == 0, s * (NMC1_A * MC1),
                     EDGES1_A + s * (NMC1_B * MC1))
    base_rows = base // IB

    def idx_copies(k, sl3):
        rs = pl.multiple_of(base_rows + k * KR1, 8)
        return (
            pltpu.make_async_copy(row2_hbm.at[pl.ds(rs, KR1)],
                                  ridx2.at[pl.ds(sl3 * KR1, KR1)], isem),
            pltpu.make_async_copy(col2_hbm.at[pl.ds(rs, KR1)],
                                  cidx2.at[pl.ds(sl3 * KR1, KR1)], isem),
            pltpu.make_async_copy(ew_hbm.at[pl.ds(base + k * MC1, MC1)],
                                  ew2.at[pl.ds(sl3 * MC1, MC1)], isem),
        )

    def issue_idx(k, sl3):
        for cp in idx_copies(k, sl3):
            cp.start()

    def drain_idx(k, sl3):
        for cp in idx_copies(k, sl3):
            cp.wait()

    def issue_gather(sl3, b):
        for j in range(KR1):
            pltpu.async_copy(zp_hbm.at[ridx2.at[sl3 * KR1 + j]],
                             val.at[pl.ds((b * KR1 + j) * IB, IB)], gsem)

    issue_idx(0, 0)
    issue_idx(1, 1)
    drain_idx(0, 0)
    issue_gather(0, 0)

    def step(k, carry):
        b = k % 2
        bn = 1 - b
        sl3 = k % 3
        sln = (k + 1) % 3

        @pl.when(k > 0)
        def _():
            for j in range(KR1):
                pltpu.make_async_copy(
                    val.at[pl.ds((bn * KR1 + j) * IB, IB)],
                    acc_sh.at[cidx2.at[((k - 1) % 3) * KR1 + j]], ssem).wait()

        @pl.when(k + 2 < nmc)
        def _():
            issue_idx(k + 2, (k + 2) % 3)

        @pl.when(k + 1 < nmc)
        def _():
            drain_idx(k + 1, sln)
            issue_gather(sln, bn)

        for j in range(KR1):
            pltpu.make_async_copy(zp_hbm.at[ridx2.at[sl3 * KR1 + j]],
                                  val.at[pl.ds((b * KR1 + j) * IB, IB)],
                                  gsem).wait()

        for j in range(MC1 // 16):
            slv = pl.ds(b * MC1 + j * 16, 16)
            sle = pl.ds(sl3 * MC1 + j * 16, 16)
            val[slv] = val[slv] * ew2[sle]

        for j in range(KR1):
            pltpu.async_copy(val.at[pl.ds((b * KR1 + j) * IB, IB)],
                             acc_sh.at[cidx2.at[sl3 * KR1 + j]], ssem, add=True)
        return carry

    lax.fori_loop(0, nmc, step, 0)
    bl = (nmc - 1) % 2
    sl_last = (nmc - 1) % 3
    for j in range(KR1):
        pltpu.make_async_copy(
            val.at[pl.ds((bl * KR1 + j) * IB, IB)],
            acc_sh.at[cidx2.at[sl_last * KR1 + j]], ssem).wait()
    _writeback(acc_sh, out_hbm, c, s)


def _mp1_call(zp, row2, col2, ew_p, z1):
    k = pl.kernel(
        _mp1_body,
        mesh=_sc_mesh(),
        out_type=jax.ShapeDtypeStruct((NC, N_PAD), jnp.float32),
        scratch_types=[
            pltpu.VMEM((3 * KR1, IB), jnp.int32),
            pltpu.VMEM((3 * KR1, IB), jnp.int32),
            pltpu.VMEM((3 * MC1,), jnp.float32),
            pltpu.VMEM((2 * MC1,), jnp.float32),
            pltpu.VMEM_SHARED((N_PAD,), jnp.float32),
            pltpu.SemaphoreType.DMA,
            pltpu.SemaphoreType.DMA,
            pltpu.SemaphoreType.DMA,
        ],
    )
    return k(zp, row2, col2, ew_p, z1)


_BN = 2000


def _tc_b_body(x_ref, w1_ref, degp_ref, hp_ref, dis_ref):
    h = jnp.dot(x_ref[...], w1_ref[...], preferred_element_type=jnp.float32)
    deg = degp_ref[:, 0] + degp_ref[:, 1] + 1.0
    dis = lax.rsqrt(deg)
    hp_ref[...] = h * dis[:, None]
    dis_ref[...] = dis[:, None]


def _b_call(x, W1p, degp_t):
    return pl.pallas_call(
        _tc_b_body,
        grid=(N_NODES // _BN,),
        in_specs=[
            pl.BlockSpec((_BN, IN_CH), lambda i: (i, 0)),
            pl.BlockSpec((IN_CH, HID_PAD), lambda i: (0, 0)),
            pl.BlockSpec((_BN, NC), lambda i: (i, 0)),
        ],
        out_specs=[
            pl.BlockSpec((_BN, HID_PAD), lambda i: (i, 0)),
            pl.BlockSpec((_BN, 1), lambda i: (i, 0)),
        ],
        out_shape=[
            jax.ShapeDtypeStruct((N_NODES, HID_PAD), jnp.float32),
            jax.ShapeDtypeStruct((N_NODES, 1), jnp.float32),
        ],
    )(x, W1p, degp_t)


def _tc_d_body(accp_ref, hp_ref, dis_ref, b1_ref, w2_ref, zp_ref):
    acc = (accp_ref[0, :, :HID_CH] + accp_ref[1, :, :HID_CH]
           + hp_ref[:, :HID_CH])
    out1 = acc * dis_ref[...] + b1_ref[...]
    h1 = jnp.maximum(out1, 0.0)
    z = jnp.dot(h1, w2_ref[...], preferred_element_type=jnp.float32)
    zp_ref[...] = z * dis_ref[...]


def _d_call(accp, hp, dis, b1r, W2):
    return pl.pallas_call(
        _tc_d_body,
        grid=(N_NODES // _BN,),
        in_specs=[
            pl.BlockSpec((NC, _BN, HID_PAD), lambda i: (0, i, 0)),
            pl.BlockSpec((_BN, HID_PAD), lambda i: (i, 0)),
            pl.BlockSpec((_BN, 1), lambda i: (i, 0)),
            pl.BlockSpec((1, HID_CH), lambda i: (0, 0)),
            pl.BlockSpec((HID_CH, 1), lambda i: (0, 0)),
        ],
        out_specs=pl.BlockSpec((_BN, 1), lambda i: (i, 0)),
        out_shape=jax.ShapeDtypeStruct((N_NODES, 1), jnp.float32),
    )(accp, hp, dis, b1r, W2)


def _tc_f_body(acc2p_ref, zp_ref, dis_ref, b2_ref, out_ref):
    a = acc2p_ref[:, 0] + acc2p_ref[:, 1]
    out_ref[...] = dis_ref[...] * (a[:, None] + zp_ref[...]) + b2_ref[...]


def _f_call(acc2p_t, zp, dis, b2r):
    return pl.pallas_call(
        _tc_f_body,
        grid=(N_NODES // _BN,),
        in_specs=[
            pl.BlockSpec((_BN, NC), lambda i: (i, 0)),
            pl.BlockSpec((_BN, 1), lambda i: (i, 0)),
            pl.BlockSpec((_BN, 1), lambda i: (i, 0)),
            pl.BlockSpec((1, 1), lambda i: (0, 0)),
        ],
        out_specs=pl.BlockSpec((_BN, 1), lambda i: (i, 0)),
        out_shape=jax.ShapeDtypeStruct((N_NODES, 1), jnp.float32),
    )(acc2p_t, zp, dis, b2r)


def kernel(x, edge_index, edge_weight, W1, b1, W2, b2):
    row = edge_index[0].astype(jnp.int32)
    col = edge_index[1].astype(jnp.int32)
    ew = edge_weight.astype(jnp.float32)
    pad = E_PAD - row.shape[0]
    row2 = jnp.concatenate([row, jnp.zeros((pad,), jnp.int32)]).reshape(-1, IB)
    col2 = jnp.concatenate([col, jnp.zeros((pad,), jnp.int32)]).reshape(-1, IB)
    ew_p = jnp.concatenate([ew, jnp.zeros((pad,), jnp.float32)])
    z1 = jnp.zeros((RPT,), jnp.float32)
    z2 = jnp.zeros((RPT, HID_PAD), jnp.float32)

    W1p = jnp.concatenate(
        [W1, jnp.zeros((IN_CH, HID_PAD - HID_CH), jnp.float32)], axis=1)
    degp = _deg_call(col2, ew_p, z1)
    hp, dis = _b_call(x, W1p, degp.T)
    accp = _mp64_call(hp, row2, col2, ew_p, z2)
    zp = _d_call(accp, hp, dis, b1.reshape(1, HID_CH), W2)
    acc2p = _mp1_call(zp.reshape(-1), row2, col2, ew_p, z1)
    out = _f_call(acc2p.T, zp, dis, b2.reshape(1, 1))
    return out.reshape(-1)


# speedup vs baseline: 1.0405x; 1.0058x over previous
import jax
import jax.numpy as jnp
from jax import lax
from jax.experimental import pallas as pl
from jax.experimental.pallas import tpu as pltpu
from jax.experimental.pallas import tpu_sc as plsc

N_NODES = 10000
IN_CH = 256
HID_CH = 64
HID_PAD = 128

NC = 2
NS = 16
NW = NC * NS
N_PAD = 10240
RPT = N_PAD // NS
E_PAD = 163840
EPW = E_PAD // NW
IB = 128

MC = 128
KR = MC // IB
NMC = EPW // MC
SUP = 8
NMC_A = 56
NMC_B = 80 - NMC_A
EDGES_A = NS * NMC_A * MC
MC1 = 1024
KR1 = MC1 // IB
NMC1 = EPW // MC1
NMC1_A = 7
NMC1_B = 10 - NMC1_A
EDGES1_A = NS * NMC1_A * MC1


def _sc_mesh():
    return plsc.VectorSubcoreMesh(core_axis_name="c", subcore_axis_name="s")


def _zero_acc_and_barrier(z_hbm, acc_sh, s):
    pltpu.sync_copy(z_hbm, acc_sh.at[pl.ds(s * RPT, RPT)])
    plsc.subcore_barrier()


def _writeback(acc_sh, out_hbm, c, s):
    plsc.subcore_barrier()
    pltpu.sync_copy(acc_sh.at[pl.ds(s * RPT, RPT)],
                    out_hbm.at[c, pl.ds(s * RPT, RPT)])


def _deg_body(col2_hbm, ew_hbm, z1_hbm, out_hbm, cidx2, ew2, acc_sh, ssem):
    c = lax.axis_index("c")
    s = lax.axis_index("s")
    wid = c * NS + s
    _zero_acc_and_barrier(z1_hbm, acc_sh, s)
    base = wid * EPW
    base_rows = wid * (EPW // IB)

    def step(k, carry):
        b = k % 2
        bn = 1 - b

        @pl.when(k > 0)
        def _():
            for j in range(KR1):
                pltpu.make_async_copy(
                    ew2.at[pl.ds((bn * KR1 + j) * IB, IB)],
                    acc_sh.at[cidx2.at[bn * KR1 + j]], ssem).wait()

        rs = base_rows + k * KR1
        pltpu.sync_copy(col2_hbm.at[pl.ds(rs, KR1)],
                        cidx2.at[pl.ds(b * KR1, KR1)])
        pltpu.sync_copy(ew_hbm.at[pl.ds(base + k * MC1, MC1)],
                        ew2.at[pl.ds(b * MC1, MC1)])
        for j in range(KR1):
            pltpu.async_copy(ew2.at[pl.ds((b * KR1 + j) * IB, IB)],
                             acc_sh.at[cidx2.at[b * KR1 + j]], ssem, add=True)
        return carry

    lax.fori_loop(0, NMC1, step, 0)
    bl = (NMC1 - 1) % 2
    for j in range(KR1):
        pltpu.make_async_copy(
            ew2.at[pl.ds((bl * KR1 + j) * IB, IB)],
            acc_sh.at[cidx2.at[bl * KR1 + j]], ssem).wait()
    _writeback(acc_sh, out_hbm, c, s)


def _deg_call(col2, ew_p, z1):
    k = pl.kernel(
        _deg_body,
        mesh=_sc_mesh(),
        out_type=jax.ShapeDtypeStruct((NC, N_PAD), jnp.float32),
        scratch_types=[
            pltpu.VMEM((2 * KR1, IB), jnp.int32),
            pltpu.VMEM((2 * MC1,), jnp.float32),
            pltpu.VMEM_SHARED((N_PAD,), jnp.float32),
            pltpu.SemaphoreType.DMA,
        ],
    )
    return k(col2, ew_p, z1)


def _mp64_body(hp_hbm, row2_hbm, col2_hbm, ew_hbm, z2_hbm, out_hbm,
               ridx2, cidx2, ew2, gdst, msg, acc_sh, gsem, ssem, isem):
    c = lax.axis_index("c")
    s = lax.axis_index("s")
    _zero_acc_and_barrier(z2_hbm, acc_sh, s)
    nmc = jnp.where(c == 0, NMC_A, NMC_B)
    base = jnp.where(c == 0, s * (NMC_A * MC),
                     EDGES_A + s * (NMC_B * MC))
    base_rows = base // IB

    def idx_copies(k, sl3):
        rs = base_rows + k
        return (
            pltpu.make_async_copy(row2_hbm.at[pl.ds(rs, 1)],
                                  ridx2.at[pl.ds(sl3, 1)], isem),
            pltpu.make_async_copy(col2_hbm.at[pl.ds(rs, 1)],
                                  cidx2.at[pl.ds(sl3, 1)], isem),
            pltpu.make_async_copy(ew_hbm.at[pl.ds(base + k * MC, MC)],
                                  ew2.at[pl.ds(sl3 * MC, MC)], isem),
        )

    def issue_idx(k, sl3):
        for cp in idx_copies(k, sl3):
            cp.start()

    def drain_idx(k, sl3):
        for cp in idx_copies(k, sl3):
            cp.wait()

    def issue_gather(sl3, b):
        pltpu.async_copy(hp_hbm.at[ridx2.at[sl3]],
                         msg.at[pl.ds(b * MC, MC)], gsem)

    issue_idx(0, 0)
    issue_idx(1, 1)
    drain_idx(0, 0)
    issue_gather(0, 0)

    def step(k, carry):
        b = k % 2
        bn = 1 - b
        sl3 = k % 3
        sln = (k + 1) % 3

        @pl.when(k > 0)
        def _():
            pltpu.make_async_copy(
                msg.at[pl.ds(bn * MC, MC)],
                acc_sh.at[cidx2.at[(k - 1) % 3]], ssem).wait()

        @pl.when(k + 2 < nmc)
        def _():
            issue_idx(k + 2, (k + 2) % 3)

        @pl.when(k + 1 < nmc)
        def _():
            drain_idx(k + 1, sln)
            issue_gather(sln, bn)

        pltpu.make_async_copy(hp_hbm.at[ridx2.at[sl3]],
                              msg.at[pl.ds(b * MC, MC)], gsem).wait()

        def sgrp(t, cc):
            ew16 = ew2[pl.ds(sl3 * MC + t * 16, 16)]
            for jj in range(16):
                w = ew16[jj]
                i = b * MC + t * 16 + jj
                for g in range(HID_CH // 16):
                    sl = pl.ds(g * 16, 16)
                    msg[i, sl] = msg[i, sl] * w
            return cc

        lax.fori_loop(0, MC // 16, sgrp, 0)

        pltpu.async_copy(msg.at[pl.ds(b * MC, MC)],
                         acc_sh.at[cidx2.at[sl3]], ssem, add=True)
        return carry

    lax.fori_loop(0, nmc, step, 0)
    bl = (nmc - 1) % 2
    pltpu.make_async_copy(
        msg.at[pl.ds(bl * MC, MC)],
        acc_sh.at[cidx2.at[(nmc - 1) % 3]], ssem).wait()
    _writeback(acc_sh, out_hbm, c, s)


def _mp64_call(hp, row2, col2, ew_p, z2):
    k = pl.kernel(
        _mp64_body,
        mesh=_sc_mesh(),
        out_type=jax.ShapeDtypeStruct((NC, N_PAD, HID_PAD), jnp.float32),
        scratch_types=[
            pltpu.VMEM((3, IB), jnp.int32),
            pltpu.VMEM((3, IB), jnp.int32),
            pltpu.VMEM((3 * MC,), jnp.float32),
            pltpu.VMEM((1, IB), jnp.int32),
            pltpu.VMEM((2 * MC, HID_PAD), jnp.float32),
            pltpu.VMEM_SHARED((N_PAD, HID_PAD), jnp.float32),
            pltpu.SemaphoreType.DMA,
            pltpu.SemaphoreType.DMA,
            pltpu.SemaphoreType.DMA,
        ],
    )
    return k(hp, row2, col2, ew_p, z2)


def _mp1_body(zp_hbm, row2_hbm, col2_hbm, ew_hbm, z1_hbm, out_hbm,
              ridx2, cidx2, ew2, val, acc_sh, gsem, ssem, isem):
    c = lax.axis_index("c")
    s = lax.axis_index("s")
    _zero_acc_and_barrier(z1_hbm, acc_sh, s)
    nmc = jnp.where(c == 0, NMC1_A, NMC1_B)
    base = jnp.where(c == 0, s * (NMC1_A * MC1),
                     EDGES1_A + s * (NMC1_B * MC1))
    base_rows = base // IB

    def idx_copies(k, sl3):
        rs = pl.multiple_of(base_rows + k * KR1, 8)
        return (
            pltpu.make_async_copy(row2_hbm.at[pl.ds(rs, KR1)],
                                  ridx2.at[pl.ds(sl3 * KR1, KR1)], isem),
            pltpu.make_async_copy(col2_hbm.at[pl.ds(rs, KR1)],
                                  cidx2.at[pl.ds(sl3 * KR1, KR1)], isem),
            pltpu.make_async_copy(ew_hbm.at[pl.ds(base + k * MC1, MC1)],
                                  ew2.at[pl.ds(sl3 * MC1, MC1)], isem),
        )

    def issue_idx(k, sl3):
        for cp in idx_copies(k, sl3):
            cp.start()

    def drain_idx(k, sl3):
        for cp in idx_copies(k, sl3):
            cp.wait()

    def issue_gather(sl3, b):
        for j in range(KR1):
            pltpu.async_copy(zp_hbm.at[ridx2.at[sl3 * KR1 + j]],
                             val.at[pl.ds((b * KR1 + j) * IB, IB)], gsem)

    issue_idx(0, 0)
    issue_idx(1, 1)
    drain_idx(0, 0)
    issue_gather(0, 0)

    def step(k, carry):
        b = k % 2
        bn = 1 - b
        sl3 = k % 3
        sln = (k + 1) % 3

        @pl.when(k > 0)
        def _():
            for j in range(KR1):
                pltpu.make_async_copy(
                    val.at[pl.ds((bn * KR1 + j) * IB, IB)],
                    acc_sh.at[cidx2.at[((k - 1) % 3) * KR1 + j]], ssem).wait()

        @pl.when(k + 2 < nmc)
        def _():
            issue_idx(k + 2, (k + 2) % 3)

        @pl.when(k + 1 < nmc)
        def _():
            drain_idx(k + 1, sln)
            issue_gather(sln, bn)

        for j in range(KR1):
            pltpu.make_async_copy(zp_hbm.at[ridx2.at[sl3 * KR1 + j]],
                                  val.at[pl.ds((b * KR1 + j) * IB, IB)],
                                  gsem).wait()

        for j in range(MC1 // 16):
            slv = pl.ds(b * MC1 + j * 16, 16)
            sle = pl.ds(sl3 * MC1 + j * 16, 16)
            val[slv] = val[slv] * ew2[sle]

        for j in range(KR1):
            pltpu.async_copy(val.at[pl.ds((b * KR1 + j) * IB, IB)],
                             acc_sh.at[cidx2.at[sl3 * KR1 + j]], ssem, add=True)
        return carry

    lax.fori_loop(0, nmc, step, 0)
    bl = (nmc - 1) % 2
    sl_last = (nmc - 1) % 3
    for j in range(KR1):
        pltpu.make_async_copy(
            val.at[pl.ds((bl * KR1 + j) * IB, IB)],
            acc_sh.at[cidx2.at[sl_last * KR1 + j]], ssem).wait()
    _writeback(acc_sh, out_hbm, c, s)


def _mp1_call(zp, row2, col2, ew_p, z1):
    k = pl.kernel(
        _mp1_body,
        mesh=_sc_mesh(),
        out_type=jax.ShapeDtypeStruct((NC, N_PAD), jnp.float32),
        scratch_types=[
            pltpu.VMEM((3 * KR1, IB), jnp.int32),
            pltpu.VMEM((3 * KR1, IB), jnp.int32),
            pltpu.VMEM((3 * MC1,), jnp.float32),
            pltpu.VMEM((2 * MC1,), jnp.float32),
            pltpu.VMEM_SHARED((N_PAD,), jnp.float32),
            pltpu.SemaphoreType.DMA,
            pltpu.SemaphoreType.DMA,
            pltpu.SemaphoreType.DMA,
        ],
    )
    return k(zp, row2, col2, ew_p, z1)


_BN = 2000


def _tc_b_body(x_ref, w1_ref, degp_ref, hp_ref, dis_ref):
    h = jnp.dot(x_ref[...], w1_ref[...], preferred_element_type=jnp.float32)
    deg = degp_ref[:, 0] + degp_ref[:, 1] + 1.0
    dis = lax.rsqrt(deg)
    hp_ref[...] = h * dis[:, None]
    dis_ref[...] = dis[:, None]


def _b_call(x, W1p, degp_t):
    return pl.pallas_call(
        _tc_b_body,
        grid=(N_NODES // _BN,),
        in_specs=[
            pl.BlockSpec((_BN, IN_CH), lambda i: (i, 0)),
            pl.BlockSpec((IN_CH, HID_PAD), lambda i: (0, 0)),
            pl.BlockSpec((_BN, NC), lambda i: (i, 0)),
        ],
        out_specs=[
            pl.BlockSpec((_BN, HID_PAD), lambda i: (i, 0)),
            pl.BlockSpec((_BN, 1), lambda i: (i, 0)),
        ],
        out_shape=[
            jax.ShapeDtypeStruct((N_NODES, HID_PAD), jnp.float32),
            jax.ShapeDtypeStruct((N_NODES, 1), jnp.float32),
        ],
    )(x, W1p, degp_t)


def _tc_d_body(accp_ref, hp_ref, dis_ref, b1_ref, w2_ref, zp_ref):
    acc = (accp_ref[0, :, :HID_CH] + accp_ref[1, :, :HID_CH]
           + hp_ref[:, :HID_CH])
    out1 = acc * dis_ref[...] + b1_ref[...]
    h1 = jnp.maximum(out1, 0.0)
    z = jnp.dot(h1, w2_ref[...], preferred_element_type=jnp.float32)
    zp_ref[...] = z * dis_ref[...]


def _d_call(accp, hp, dis, b1r, W2):
    return pl.pallas_call(
        _tc_d_body,
        grid=(N_NODES // _BN,),
        in_specs=[
            pl.BlockSpec((NC, _BN, HID_PAD), lambda i: (0, i, 0)),
            pl.BlockSpec((_BN, HID_PAD), lambda i: (i, 0)),
            pl.BlockSpec((_BN, 1), lambda i: (i, 0)),
            pl.BlockSpec((1, HID_CH), lambda i: (0, 0)),
            pl.BlockSpec((HID_CH, 1), lambda i: (0, 0)),
        ],
        out_specs=pl.BlockSpec((_BN, 1), lambda i: (i, 0)),
        out_shape=jax.ShapeDtypeStruct((N_NODES, 1), jnp.float32),
    )(accp, hp, dis, b1r, W2)


def _tc_f_body(acc2p_ref, zp_ref, dis_ref, b2_ref, out_ref):
    a = acc2p_ref[:, 0] + acc2p_ref[:, 1]
    out_ref[...] = dis_ref[...] * (a[:, None] + zp_ref[...]) + b2_ref[...]


def _f_call(acc2p_t, zp, dis, b2r):
    return pl.pallas_call(
        _tc_f_body,
        grid=(N_NODES // _BN,),
        in_specs=[
            pl.BlockSpec((_BN, NC), lambda i: (i, 0)),
            pl.BlockSpec((_BN, 1), lambda i: (i, 0)),
            pl.BlockSpec((_BN, 1), lambda i: (i, 0)),
            pl.BlockSpec((1, 1), lambda i: (0, 0)),
        ],
        out_specs=pl.BlockSpec((_BN, 1), lambda i: (i, 0)),
        out_shape=jax.ShapeDtypeStruct((N_NODES, 1), jnp.float32),
    )(acc2p_t, zp, dis, b2r)


def kernel(x, edge_index, edge_weight, W1, b1, W2, b2):
    row = edge_index[0].astype(jnp.int32)
    col = edge_index[1].astype(jnp.int32)
    ew = edge_weight.astype(jnp.float32)
    pad = E_PAD - row.shape[0]
    row2 = jnp.concatenate([row, jnp.zeros((pad,), jnp.int32)]).reshape(-1, IB)
    col2 = jnp.concatenate([col, jnp.zeros((pad,), jnp.int32)]).reshape(-1, IB)
    ew_p = jnp.concatenate([ew, jnp.zeros((pad,), jnp.float32)])
    z1 = jnp.zeros((RPT,), jnp.float32)
    z2 = jnp.zeros((RPT, HID_PAD), jnp.float32)

    W1p = jnp.concatenate(
        [W1, jnp.zeros((IN_CH, HID_PAD - HID_CH), jnp.float32)], axis=1)
    degp = _deg_call(col2, ew_p, z1)
    hp, dis = _b_call(x, W1p, degp.T)
    accp = _mp64_call(hp, row2, col2, ew_p, z2)
    zp = _d_call(accp, hp, dis, b1.reshape(1, HID_CH), W2)
    acc2p = _mp1_call(zp.reshape(-1), row2, col2, ew_p, z1)
    out = _f_call(acc2p.T, zp, dis, b2.reshape(1, 1))
    return out.reshape(-1)
